# Initial kernel scaffold; baseline (speedup 1.0000x reference)
import jax
import jax.numpy as jnp
from jax.experimental import pallas as pl


def kernel(x, n_pts, bbox, W1, b1, g1, be1, W2, b2, g2, be2, W3, b3, g3, be3, W4, b4, g4, be4, bW1, bb1, bg1, bbe1, bW2, bb2, bg2, bbe2, bpW, bpb, bpg, bpbe, pW, pb, pg, pbe):
    raise NotImplementedError("write your pallas kernel here")


# speedup vs baseline: 6.5567x; 6.5567x over previous
import functools

import jax
import jax.numpy as jnp
from jax import lax
from jax.experimental import pallas as pl

_SELU_SCALE = 1.0507009873554805
_SELU_ALPHA = 1.6732632423543772

_HI = jax.lax.Precision.HIGHEST


def _layer_norm(y, g, b, eps=1e-5):
    m = jnp.mean(y, axis=-1, keepdims=True)
    v = jnp.mean((y - m) * (y - m), axis=-1, keepdims=True)
    return (y - m) * lax.rsqrt(v + eps) * g + b


def _selu(y):
    return _SELU_SCALE * jnp.where(y > 0, y, _SELU_ALPHA * (jnp.exp(y) - 1.0))


def _edgeconv_tile(xt, xb, wa, wd, b, g, be, k, n, tile):
    proj = lax.dot_general(xb, wa, (((1,), (0,)), ((), ())), precision=_HI)
    base = lax.dot_general(xt, wd, (((1,), (0,)), ((), ())), precision=_HI) + b

    gram = lax.dot_general(xt, xb, (((1,), (1,)), ((), ())), precision=_HI)
    d_t = jnp.sum(xt * xt, axis=1, keepdims=True)
    ones = jnp.ones((1, xb.shape[1]), jnp.float32)
    d_row = lax.dot_general(ones, xb * xb, (((1,), (1,)), ((), ())),
                            precision=_HI)
    work = d_t + d_row - 2.0 * gram

    jcol = lax.broadcasted_iota(jnp.int32, (tile, n), 1)
    acc = jnp.full((tile, 64), -jnp.inf, jnp.float32)
    for _ in range(k):
        m = jnp.min(work, axis=1, keepdims=True)
        first = jnp.min(jnp.where(work == m, jcol, n), axis=1, keepdims=True)
        hot = jcol == first
        work = jnp.where(hot, jnp.inf, work)
        oh = hot.astype(jnp.float32)
        neigh = lax.dot_general(oh, proj, (((1,), (0,)), ((), ())),
                                precision=_HI)
        acc = jnp.maximum(acc, _selu(_layer_norm(neigh + base, g, be)))
    return acc


def _layer_body(xt_ref, x_ref, wa_ref, wd_ref, b_ref, g_ref, be_ref,
                out_ref, pool_ref, *, n, tile, k):
    xb = x_ref[0]
    xt = xt_ref[0]
    acc = _edgeconv_tile(xt, xb, wa_ref[...], wd_ref[...],
                         b_ref[...], g_ref[...], be_ref[...], k, n, tile)
    out_ref[0] = acc
    pooled = jnp.max(acc, axis=0, keepdims=True)
    pool_ref[0] = jnp.broadcast_to(pooled, (8, 64))


def _edgeconv_layer(x, W, b, g, be, k, tile):
    B, N, C = x.shape
    wa = W[:C]
    wd = W[C:] - W[:C]
    nt = N // tile
    body = functools.partial(_layer_body, n=N, tile=tile, k=k)
    out, pooled = pl.pallas_call(
        body,
        grid=(B, nt),
        in_specs=[
            pl.BlockSpec((1, tile, C), lambda b_, t_: (b_, t_, 0)),
            pl.BlockSpec((1, N, C), lambda b_, t_: (b_, 0, 0)),
            pl.BlockSpec((C, 64), lambda b_, t_: (0, 0)),
            pl.BlockSpec((C, 64), lambda b_, t_: (0, 0)),
            pl.BlockSpec((1, 64), lambda b_, t_: (0, 0)),
            pl.BlockSpec((1, 64), lambda b_, t_: (0, 0)),
            pl.BlockSpec((1, 64), lambda b_, t_: (0, 0)),
        ],
        out_specs=[
            pl.BlockSpec((1, tile, 64), lambda b_, t_: (b_, t_, 0)),
            pl.BlockSpec((1, 8, 64), lambda b_, t_: (b_, t_, 0)),
        ],
        out_shape=[
            jax.ShapeDtypeStruct((B, N, 64), jnp.float32),
            jax.ShapeDtypeStruct((B, nt * 8, 64), jnp.float32),
        ],
    )(x, x, wa, wd, b.reshape(1, 64), g.reshape(1, 64), be.reshape(1, 64))
    return out, pooled


def _bbox_body(x_ref, wa1_ref, wd1_ref, b1_ref, g1_ref, be1_ref,
               wa2_ref, wd2_ref, b2_ref, g2_ref, be2_ref,
               x1_ref, x2_ref):
    xb = x_ref[0]
    x1 = _edgeconv_tile(xb, xb, wa1_ref[...], wd1_ref[...], b1_ref[...],
                        g1_ref[...], be1_ref[...], 4, 8, 8)
    x2 = _edgeconv_tile(x1, x1, wa2_ref[...], wd2_ref[...], b2_ref[...],
                        g2_ref[...], be2_ref[...], 4, 8, 8)
    x1_ref[0] = x1
    x2_ref[0] = x2


def _bbox_branch(bbox, bW1, bb1, bg1, bbe1, bW2, bb2, bg2, bbe2):
    B = bbox.shape[0]
    wa1, wd1 = bW1[:3], bW1[3:] - bW1[:3]
    wa2, wd2 = bW2[:64], bW2[64:] - bW2[:64]
    r = lambda v: v.reshape(1, 64)
    x1, x2 = pl.pallas_call(
        _bbox_body,
        grid=(B,),
        in_specs=[
            pl.BlockSpec((1, 8, 3), lambda b_: (b_, 0, 0)),
            pl.BlockSpec((3, 64), lambda b_: (0, 0)),
            pl.BlockSpec((3, 64), lambda b_: (0, 0)),
            pl.BlockSpec((1, 64), lambda b_: (0, 0)),
            pl.BlockSpec((1, 64), lambda b_: (0, 0)),
            pl.BlockSpec((1, 64), lambda b_: (0, 0)),
            pl.BlockSpec((64, 64), lambda b_: (0, 0)),
            pl.BlockSpec((64, 64), lambda b_: (0, 0)),
            pl.BlockSpec((1, 64), lambda b_: (0, 0)),
            pl.BlockSpec((1, 64), lambda b_: (0, 0)),
            pl.BlockSpec((1, 64), lambda b_: (0, 0)),
        ],
        out_specs=[
            pl.BlockSpec((1, 8, 64), lambda b_: (b_, 0, 0)),
            pl.BlockSpec((1, 8, 64), lambda b_: (b_, 0, 0)),
        ],
        out_shape=[
            jax.ShapeDtypeStruct((B, 8, 64), jnp.float32),
            jax.ShapeDtypeStruct((B, 8, 64), jnp.float32),
        ],
    )(bbox, wa1, wd1, r(bb1), r(bg1), r(bbe1),
      wa2, wd2, r(bb2), r(bg2), r(bbe2))
    return x1, x2


def _head_body(p1_ref, p2_ref, p3_ref, p4_ref, x1_ref, x2_ref,
               bpw1_ref, bpw2_ref, bpb_ref, bpg_ref, bpbe_ref,
               pw1_ref, pw2_ref, pw3_ref, pw4_ref, pw5_ref,
               pb_ref, pg_ref, pbe_ref, out_ref):
    p1 = jnp.max(p1_ref[...], axis=1)
    p2 = jnp.max(p2_ref[...], axis=1)
    p3 = jnp.max(p3_ref[...], axis=1)
    p4 = jnp.max(p4_ref[...], axis=1)
    x1m = jnp.max(x1_ref[...], axis=1)
    x2m = jnp.max(x2_ref[...], axis=1)

    mm = lambda a, w: lax.dot_general(a, w, (((1,), (0,)), ((), ())),
                                      precision=_HI)
    fb = mm(x1m, bpw1_ref[...]) + mm(x2m, bpw2_ref[...]) + bpb_ref[...]
    fb = _selu(_layer_norm(fb, bpg_ref[...], bpbe_ref[...]))

    f = (mm(p1, pw1_ref[...]) + mm(p2, pw2_ref[...]) + mm(p3, pw3_ref[...])
         + mm(p4, pw4_ref[...]) + mm(fb, pw5_ref[...]) + pb_ref[...])
    f = _layer_norm(f, pg_ref[...], pbe_ref[...])
    nrm = jnp.sqrt(jnp.sum(f * f, axis=1, keepdims=True))
    out_ref[...] = f / (nrm + 1e-9)


def _head(p1, p2, p3, p4, x1, x2, bpW, bpb, bpg, bpbe, pW, pb, pg, pbe):
    B = p1.shape[0]
    r64 = lambda v: v.reshape(1, 64)
    r128 = lambda v: v.reshape(1, 128)
    full = lambda a: pl.BlockSpec(a.shape, lambda: tuple(0 for _ in a.shape))
    args = (p1, p2, p3, p4, x1, x2,
            bpW[:64], bpW[64:], r64(bpb), r64(bpg), r64(bpbe),
            pW[0:64], pW[64:128], pW[128:192], pW[192:256], pW[256:320],
            r128(pb), r128(pg), r128(pbe))
    return pl.pallas_call(
        _head_body,
        grid=(),
        in_specs=[full(a) for a in args],
        out_specs=pl.BlockSpec((B, 128), lambda: (0, 0)),
        out_shape=jax.ShapeDtypeStruct((B, 128), jnp.float32),
    )(*args)


def kernel(x, n_pts, bbox, W1, b1, g1, be1, W2, b2, g2, be2, W3, b3, g3, be3,
           W4, b4, g4, be4, bW1, bb1, bg1, bbe1, bW2, bb2, bg2, bbe2,
           bpW, bpb, bpg, bpbe, pW, pb, pg, pbe):
    B = n_pts.shape[0]
    P = x.shape[0] // B
    xb = x.reshape(B, P, 3)
    f1, p1 = _edgeconv_layer(xb, W1, b1, g1, be1, k=10, tile=256)
    f2, p2 = _edgeconv_layer(f1, W2, b2, g2, be2, k=10, tile=256)
    f3, p3 = _edgeconv_layer(f2, W3, b3, g3, be3, k=10, tile=256)
    f4, p4 = _edgeconv_layer(f3, W4, b4, g4, be4, k=10, tile=256)
    x1, x2 = _bbox_branch(bbox, bW1, bb1, bg1, bbe1, bW2, bb2, bg2, bbe2)
    return _head(p1, p2, p3, p4, x1, x2, bpW, bpb, bpg, bpbe, pW, pb, pg, pbe)
